# baseline (speedup 1.0000x reference)
import functools

import jax
import jax.numpy as jnp
from jax import lax
from jax.experimental import pallas as pl
from jax.experimental.pallas import tpu as pltpu
from jax.experimental.pallas import tpu_sc as plsc

_K = 128
_L = 16


def _imin():
    return jnp.int32(-(2 ** 31))


def _encode_body(x_ref, w_ref, b_ref, out_ref):
    out_ref[...] = (
        jnp.dot(x_ref[...], w_ref[...], preferred_element_type=jnp.float32)
        + b_ref[...]
    )


def _scan_hist(hist_v, nbuckets, rem):
    nch = nbuckets // _L

    def cond(st):
        j, carry, found, bkt, remw = st
        return jnp.logical_and(found == 0, j >= 0)

    def body(st):
        j, carry, found, bkt, remw = st
        h = hist_v[pl.ds(j * _L, _L)]
        hr = lax.rev(h, (0,))
        cs = jnp.cumsum(hr)
        csum = carry + cs
        hit = csum >= rem
        anyhit = jnp.max(hit.astype(jnp.int32))
        io = lax.iota(jnp.int32, 16)
        l = jnp.min(jnp.where(hit, io, jnp.int32(16)))
        above = jnp.min(jnp.where(hit, csum - hr, jnp.int32(2 ** 30)))
        tot = carry + jnp.sum(h)
        newbkt = jnp.where(anyhit == 1, j * _L + (_L - 1) - l, bkt)
        newremw = jnp.where(anyhit == 1, rem - above, remw)
        newcarry = jnp.where(anyhit == 1, carry, tot)
        return (j - 1, newcarry, anyhit, newbkt, newremw)

    st0 = (jnp.int32(nch - 1), jnp.int32(0), jnp.int32(0), jnp.int32(0), rem)
    _, _, _, bkt, remw = lax.while_loop(cond, body, st0)
    return bkt, remw


def _sc_recur_body(pre_hbm, gate_hbm, z_hbm, pre_v, z_v, key_v, gate_v, hist_v):
    B, T, S = pre_hbm.shape
    nchunks = S // _L
    cid = lax.axis_index("c")
    sid = lax.axis_index("s")
    wid = sid * 2 + cid

    @pl.when(wid < B)
    def _():
        b = wid
        pltpu.sync_copy(gate_hbm, gate_v)

        def zinit(i, _):
            z_v[pl.ds(i * _L, _L)] = jnp.zeros((_L,), jnp.float32)
            return 0

        lax.fori_loop(0, nchunks, zinit, 0)

        def tstep(t, _):
            pltpu.sync_copy(pre_hbm.at[b, t], pre_v)

            def hzero(i, _):
                hist_v[pl.ds(i * _L, _L)] = jnp.zeros((_L,), jnp.int32)
                return 0

            lax.fori_loop(0, 2048 // _L, hzero, 0)

            def p1(i, _):
                sl = pl.ds(i * _L, _L)
                p = pre_v[sl] + gate_v[sl] * z_v[sl]
                pre_v[sl] = p
                bb = lax.bitcast_convert_type(p, jnp.int32)
                u = jnp.where(bb >= 0, bb ^ _imin(), ~bb)
                key_v[sl] = u
                bkt = lax.shift_right_logical(u, 21)
                plsc.addupdate_scatter(hist_v, [bkt], jnp.ones((_L,), jnp.int32))
                return 0

            lax.fori_loop(0, nchunks, p1, 0)
            b1, rem1 = _scan_hist(hist_v, 2048, jnp.int32(_K))

            lax.fori_loop(0, 2048 // _L, hzero, 0)

            def p2(i, _):
                sl = pl.ds(i * _L, _L)
                u = key_v[sl]
                m = lax.shift_right_logical(u, 21) == b1
                bkt = lax.shift_right_logical(u, 10) & jnp.int32(2047)
                plsc.addupdate_scatter(
                    hist_v, [bkt], jnp.ones((_L,), jnp.int32), mask=m
                )
                return 0

            lax.fori_loop(0, nchunks, p2, 0)
            b2, rem2 = _scan_hist(hist_v, 2048, rem1)
            pfx2 = (b1 << 11) | b2

            lax.fori_loop(0, 1024 // _L, hzero, 0)

            def p3(i, _):
                sl = pl.ds(i * _L, _L)
                u = key_v[sl]
                m = lax.shift_right_logical(u, 10) == pfx2
                bkt = u & jnp.int32(1023)
                plsc.addupdate_scatter(
                    hist_v, [bkt], jnp.ones((_L,), jnp.int32), mask=m
                )
                return 0

            lax.fori_loop(0, nchunks, p3, 0)
            b3, _unused = _scan_hist(hist_v, 1024, rem2)
            tau_s = ((pfx2 << 10) | b3) ^ _imin()

            def pz(i, _):
                sl = pl.ds(i * _L, _L)
                keep = (key_v[sl] ^ _imin()) >= tau_s
                z_v[sl] = jnp.where(
                    keep, jnp.maximum(pre_v[sl], 0.0), 0.0
                )
                return 0

            lax.fori_loop(0, nchunks, pz, 0)
            pltpu.sync_copy(z_v, z_hbm.at[b, t])
            return 0

        lax.fori_loop(0, T, tstep, 0)


def _sc_recur(pre3, gate1d):
    B, T, S = pre3.shape
    mesh = plsc.VectorSubcoreMesh(core_axis_name="c", subcore_axis_name="s")
    fn = pl.kernel(
        _sc_recur_body,
        mesh=mesh,
        compiler_params=pltpu.CompilerParams(needs_layout_passes=False),
        out_type=jax.ShapeDtypeStruct((B, T, S), jnp.float32),
        scratch_types=[
            pltpu.VMEM((S,), jnp.float32),
            pltpu.VMEM((S,), jnp.float32),
            pltpu.VMEM((S,), jnp.int32),
            pltpu.VMEM((S,), jnp.float32),
            pltpu.VMEM((2048,), jnp.int32),
        ],
    )
    return fn(pre3, gate1d)


def _decode_body(z_ref, w_ref, b_ref, x_ref, xhat_ref, loss_ref, *, nk, inv_bt):
    k = pl.program_id(0)
    part = jnp.dot(z_ref[...], w_ref[...], preferred_element_type=jnp.float32)

    @pl.when(k == 0)
    def _():
        xhat_ref[...] = part

    @pl.when(k > 0)
    def _():
        xhat_ref[...] = xhat_ref[...] + part

    @pl.when(k == nk - 1)
    def _():
        xh = xhat_ref[...] + b_ref[...]
        xhat_ref[...] = xh
        d = xh - x_ref[...]
        loss_ref[0, 0] = jnp.sum(d * d) * inv_bt


def kernel(x, W_enc, W_dec, b_enc, b_dec, gate_raw):
    B, T, D_IN = x.shape
    D_SAE = W_enc.shape[1]
    BT = B * T

    x2 = x.reshape(BT, D_IN)
    gate1d = jax.nn.sigmoid(gate_raw)
    b_enc2 = b_enc.reshape(1, D_SAE)
    b_dec2 = b_dec.reshape(1, D_IN)

    SN = 2048
    pre2 = pl.pallas_call(
        _encode_body,
        grid=(D_SAE // SN,),
        in_specs=[
            pl.BlockSpec((BT, D_IN), lambda j: (0, 0)),
            pl.BlockSpec((D_IN, SN), lambda j: (0, j)),
            pl.BlockSpec((1, SN), lambda j: (0, j)),
        ],
        out_specs=pl.BlockSpec((BT, SN), lambda j: (0, j)),
        out_shape=jax.ShapeDtypeStruct((BT, D_SAE), jnp.float32),
    )(x2, W_enc, b_enc2)

    z_seq = _sc_recur(pre2.reshape(B, T, D_SAE), gate1d)

    SK = 1024
    NK = D_SAE // SK
    xhat2, loss = pl.pallas_call(
        functools.partial(_decode_body, nk=NK, inv_bt=1.0 / BT),
        grid=(NK,),
        in_specs=[
            pl.BlockSpec((BT, SK), lambda k: (0, k)),
            pl.BlockSpec((SK, D_IN), lambda k: (k, 0)),
            pl.BlockSpec((1, D_IN), lambda k: (0, 0)),
            pl.BlockSpec((BT, D_IN), lambda k: (0, 0)),
        ],
        out_specs=[
            pl.BlockSpec((BT, D_IN), lambda k: (0, 0)),
            pl.BlockSpec(memory_space=pltpu.SMEM),
        ],
        out_shape=[
            jax.ShapeDtypeStruct((BT, D_IN), jnp.float32),
            jax.ShapeDtypeStruct((1, 1), jnp.float32),
        ],
    )(z_seq.reshape(BT, D_SAE), W_dec, b_dec2, x2)

    return (loss[0, 0], xhat2.reshape(B, T, D_IN), z_seq[:, -1, :])
